# Initial kernel scaffold; baseline (speedup 1.0000x reference)
import jax
import jax.numpy as jnp
from jax.experimental import pallas as pl


def kernel(x, emb):
    raise NotImplementedError("write your pallas kernel here")


# speedup vs baseline: 2.2909x; 2.2909x over previous
import jax
import jax.numpy as jnp
from jax.experimental import pallas as pl


def kernel(x, emb):
    B, L = x.shape[0], x.shape[1]
    D = emb.shape[1]
    BLK = 512

    def body(emb_ref, o_ref):
        o_ref[...] = jnp.broadcast_to(emb_ref[...][None], (B, BLK, D))

    return pl.pallas_call(
        body,
        grid=(L // BLK,),
        in_specs=[pl.BlockSpec((BLK, D), lambda i: (i, 0))],
        out_specs=pl.BlockSpec((B, BLK, D), lambda i: (0, i, 0)),
        out_shape=jax.ShapeDtypeStruct((B, L, D), jnp.float32),
    )(emb)
